# baseline (speedup 1.0000x reference)
import functools

import jax
import jax.numpy as jnp
from jax import lax
from jax.experimental import pallas as pl
from jax.experimental.pallas import tpu as pltpu
from jax.experimental.pallas import tpu_sc as plsc

_N = 10000
_E = 320000
_D = 128
_C = 40
_L = 3
_EPS = 1e-5

_NC = 2
_NS = 16
_NW = _NC * _NS
_EPW = _E // _NW
_K = 125
_NCHUNK = _EPW // _K
_NP = 10240
_RPT = _NP // _NS

_mesh = plsc.VectorSubcoreMesh(core_axis_name="c", subcore_axis_name="s")

_ZC = 80


@functools.partial(
    pl.kernel,
    out_type=jax.ShapeDtypeStruct((_NC, _NP, _D), jnp.float32),
    mesh=_mesh,
    scratch_types=[
        pltpu.VMEM((_NCHUNK, _K), jnp.int32),
        pltpu.VMEM((_K, _D), jnp.float32),
        pltpu.VMEM_SHARED((_NP, _D), jnp.float32),
        pltpu.SemaphoreType.DMA,
    ],
)
def _sc_degree(colw_hbm, dummy_hbm, out_hbm, idx_v, ones_v, acc_sh, sem):
    c = lax.axis_index("c")
    s = lax.axis_index("s")
    w = c * _NS + s

    pltpu.async_copy(colw_hbm.at[w], idx_v, sem)

    def _fill(val):
        def _f(i, carry):
            for j in range(_D // 16):
                ones_v[i, pl.ds(j * 16, 16)] = jnp.full((16,), val, jnp.float32)
            return carry
        return _f

    lax.fori_loop(0, _ZC, _fill(0.0), 0)

    def _zero_acc(k, carry):
        pltpu.sync_copy(ones_v.at[pl.ds(0, _ZC)],
                        acc_sh.at[pl.ds(s * _RPT + k * _ZC, _ZC)])
        return carry

    lax.fori_loop(0, _RPT // _ZC, _zero_acc, 0)
    lax.fori_loop(0, _K, _fill(1.0), 0)
    pltpu.make_async_copy(colw_hbm.at[w], idx_v, sem).wait()
    plsc.subcore_barrier()

    _W = 8

    def _step(i, carry):
        pltpu.async_copy(ones_v, acc_sh.at[idx_v.at[i]], sem, add=True)

        @pl.when(i >= _W)
        def _():
            pltpu.make_async_copy(dummy_hbm, ones_v, sem).wait()

        return carry

    lax.fori_loop(0, _NCHUNK, _step, 0)

    def _drain(i, carry):
        pltpu.make_async_copy(dummy_hbm, ones_v, sem).wait()
        return carry

    lax.fori_loop(0, _W, _drain, 0)
    plsc.subcore_barrier()
    pltpu.sync_copy(
        acc_sh.at[pl.ds(s * _RPT, _RPT)], out_hbm.at[c, pl.ds(s * _RPT, _RPT)]
    )


_SCH = 8
_NSEG = _NCHUNK // _SCH


@functools.partial(
    pl.kernel,
    out_type=jax.ShapeDtypeStruct((_NC, _NP, _D), jnp.float32),
    mesh=_mesh,
    scratch_types=[
        pltpu.VMEM((2, _SCH, _K), jnp.int32),
        pltpu.VMEM((2, _SCH, _K), jnp.int32),
        pltpu.VMEM((_K, _D), jnp.float32),
        pltpu.VMEM((_K, _D), jnp.float32),
        pltpu.VMEM_SHARED((_NP, _D), jnp.float32),
        pltpu.SemaphoreType.DMA,
        pltpu.SemaphoreType.DMA,
        pltpu.SemaphoreType.DMA,
        pltpu.SemaphoreType.DMA,
        pltpu.SemaphoreType.DMA,
    ],
)
def _sc_aggregate(hp_hbm, roww_hbm, colw_hbm, dummy_hbm, out_hbm, ri_v, ci_v,
                  buf0, buf1, acc_sh, sg0, sg1, ss0, ss1, si):
    c = lax.axis_index("c")
    s = lax.axis_index("s")
    w = c * _NS + s

    pltpu.async_copy(roww_hbm.at[w, pl.ds(0, _SCH)], ri_v.at[0], si)
    pltpu.async_copy(colw_hbm.at[w, pl.ds(0, _SCH)], ci_v.at[0], si)

    def _fz(i, carry):
        for j in range(_D // 16):
            buf0[i, pl.ds(j * 16, 16)] = jnp.zeros((16,), jnp.float32)
        return carry

    lax.fori_loop(0, _ZC, _fz, 0)

    def _zero_acc(k, carry):
        pltpu.sync_copy(buf0.at[pl.ds(0, _ZC)],
                        acc_sh.at[pl.ds(s * _RPT + k * _ZC, _ZC)])
        return carry

    lax.fori_loop(0, _RPT // _ZC, _zero_acc, 0)
    pltpu.make_async_copy(roww_hbm.at[w, pl.ds(0, _SCH)], ri_v.at[0], si).wait()
    pltpu.make_async_copy(colw_hbm.at[w, pl.ds(0, _SCH)], ci_v.at[0], si).wait()
    plsc.subcore_barrier()

    def _wait_gather(buf, sg):
        pltpu.make_async_copy(dummy_hbm, buf, sg).wait()

    def _drain_scatter(buf, ss):
        pltpu.make_async_copy(dummy_hbm, buf, ss).wait()

    def _seg(h, carry):
        par = lax.rem(h, 2)
        pnext = 1 - par

        @pl.when(h + 1 < _NSEG)
        def _():
            pltpu.async_copy(
                roww_hbm.at[w, pl.ds((h + 1) * _SCH, _SCH)], ri_v.at[pnext], si)
            pltpu.async_copy(
                colw_hbm.at[w, pl.ds((h + 1) * _SCH, _SCH)], ci_v.at[pnext], si)

        for j0 in range(0, _SCH, 2):
            j1 = j0 + 1
            if j0 >= 2:
                _drain_scatter(buf0, ss0)
            else:
                @pl.when(h > 0)
                def _():
                    _drain_scatter(buf0, ss0)
            pltpu.async_copy(hp_hbm.at[ri_v.at[par, j0]], buf0, sg0)
            if j0 >= 2:
                _drain_scatter(buf1, ss1)
            else:
                @pl.when(h > 0)
                def _():
                    _drain_scatter(buf1, ss1)
            pltpu.async_copy(hp_hbm.at[ri_v.at[par, j1]], buf1, sg1)
            _wait_gather(buf0, sg0)
            pltpu.async_copy(buf0, acc_sh.at[ci_v.at[par, j0]], ss0, add=True)
            _wait_gather(buf1, sg1)
            pltpu.async_copy(buf1, acc_sh.at[ci_v.at[par, j1]], ss1, add=True)

        @pl.when(h + 1 < _NSEG)
        def _():
            pltpu.make_async_copy(
                roww_hbm.at[w, pl.ds(0, _SCH)], ri_v.at[pnext], si).wait()
            pltpu.make_async_copy(
                colw_hbm.at[w, pl.ds(0, _SCH)], ci_v.at[pnext], si).wait()

        return carry

    lax.fori_loop(0, _NSEG, _seg, 0)
    _drain_scatter(buf0, ss0)
    _drain_scatter(buf1, ss1)
    plsc.subcore_barrier()
    pltpu.sync_copy(
        acc_sh.at[pl.ds(s * _RPT, _RPT)], out_hbm.at[c, pl.ds(s * _RPT, _RPT)]
    )


def _pre_body(p_ref, x_ref, we_ref, be_ref, w0_ref, dv_ref, hp_ref):
    d = p_ref[0, :_N] + p_ref[1, :_N]
    dv = jnp.where(d > 0, lax.rsqrt(d), 0.0)
    dv_ref[...] = dv
    h = lax.dot_general(x_ref[...], we_ref[...], (((1,), (1,)), ((), ())),
                        preferred_element_type=jnp.float32)
    h = jnp.maximum(h + be_ref[...], 0.0)
    hw = lax.dot_general(h, w0_ref[...], (((1,), (1,)), ((), ())),
                         preferred_element_type=jnp.float32)
    hp_ref[...] = hw * dv


def _pre_call(parts, x, we, be, w0):
    return pl.pallas_call(
        _pre_body,
        out_shape=[
            jax.ShapeDtypeStruct((_N, _D), jnp.float32),
            jax.ShapeDtypeStruct((_N, _D), jnp.float32),
        ],
    )(parts, x, we, be, w0)


def _layer_mm_body(p_ref, dv_ref, b_ref, r_ref, g_ref, be_ref, w_ref,
                   y_ref, hp_ref):
    y = (p_ref[0, :_N] + p_ref[1, :_N]) * dv_ref[...] + b_ref[...] + r_ref[...]
    y_ref[...] = y
    mu = jnp.mean(y, axis=0, keepdims=True)
    var = jnp.mean(y * y, axis=0, keepdims=True) - mu * mu
    h = jnp.maximum(
        (y - mu) * lax.rsqrt(var + _EPS) * g_ref[...] + be_ref[...], 0.0)
    hw = lax.dot_general(h, w_ref[...], (((1,), (1,)), ((), ())),
                         preferred_element_type=jnp.float32)
    hp_ref[...] = hw * dv_ref[...]


def _layer_mm0_body(p_ref, dv_ref, b_ref, g_ref, be_ref, w_ref, y_ref, hp_ref):
    y = (p_ref[0, :_N] + p_ref[1, :_N]) * dv_ref[...] + b_ref[...]
    y_ref[...] = y
    mu = jnp.mean(y, axis=0, keepdims=True)
    var = jnp.mean(y * y, axis=0, keepdims=True) - mu * mu
    h = jnp.maximum(
        (y - mu) * lax.rsqrt(var + _EPS) * g_ref[...] + be_ref[...], 0.0)
    hw = lax.dot_general(h, w_ref[...], (((1,), (1,)), ((), ())),
                         preferred_element_type=jnp.float32)
    hp_ref[...] = hw * dv_ref[...]


def _layer_call(parts, dinv, b, res, g, be, w_next):
    out_shape = [
        jax.ShapeDtypeStruct((_N, _D), jnp.float32),
        jax.ShapeDtypeStruct((_N, _D), jnp.float32),
    ]
    if res is None:
        return pl.pallas_call(_layer_mm0_body, out_shape=out_shape)(
            parts, dinv, b, g, be, w_next)
    return pl.pallas_call(_layer_mm_body, out_shape=out_shape)(
        parts, dinv, b, res, g, be, w_next)


def _layer_pred_body(p_ref, dv_ref, b_ref, r_ref, g_ref, be_ref, w_ref,
                     bp_ref, o_ref):
    y = (p_ref[0, :_N] + p_ref[1, :_N]) * dv_ref[...] + b_ref[...] + r_ref[...]
    mu = jnp.mean(y, axis=0, keepdims=True)
    var = jnp.mean(y * y, axis=0, keepdims=True) - mu * mu
    h = jnp.maximum(
        (y - mu) * lax.rsqrt(var + _EPS) * g_ref[...] + be_ref[...], 0.0)
    hw = lax.dot_general(h, w_ref[...], (((1,), (1,)), ((), ())),
                         preferred_element_type=jnp.float32)
    o_ref[...] = hw + bp_ref[...]


def _layer_pred_call(parts, dinv, b, res, g, be, wp, bp):
    return pl.pallas_call(
        _layer_pred_body,
        out_shape=jax.ShapeDtypeStruct((_N, _D), jnp.float32),
    )(parts, dinv, b, res, g, be, wp, bp)


def kernel(x, edge_index, W_enc, b_enc, Ws, bs, gammas, betas, W_pred, b_pred):
    roww = edge_index[0].reshape(_NW, _NCHUNK, _K)
    colw = edge_index[1].reshape(_NW, _NCHUNK, _K)

    dummy = jnp.zeros((_K, _D), jnp.float32)
    deg_parts = _sc_degree(colw, dummy)
    dinv, hp = _pre_call(deg_parts, x, W_enc, b_enc.reshape(1, _D), Ws[0])

    wp_pad = jnp.zeros((_D, _D), jnp.float32).at[:_C].set(W_pred)
    bp_pad = jnp.zeros((1, _D), jnp.float32).at[0, :_C].set(b_pred)

    y_last = None
    out = None
    for i in range(_L):
        parts = _sc_aggregate(hp, roww, colw, dummy)
        b = bs[i].reshape(1, _D)
        g = gammas[i].reshape(1, _D)
        be = betas[i].reshape(1, _D)
        if i < _L - 1:
            y, hp = _layer_call(parts, dinv, b, y_last, g, be, Ws[i + 1])
            y_last = y
        else:
            out = _layer_pred_call(parts, dinv, b, y_last, g, be, wp_pad,
                                   bp_pad)
    return out[:, :_C]


# speedup vs baseline: 1.0004x; 1.0004x over previous
import functools

import jax
import jax.numpy as jnp
from jax import lax
from jax.experimental import pallas as pl
from jax.experimental.pallas import tpu as pltpu
from jax.experimental.pallas import tpu_sc as plsc

_N = 10000
_E = 320000
_D = 128
_C = 40
_L = 3
_EPS = 1e-5

_NC = 2
_NS = 16
_NW = _NC * _NS
_EPW = _E // _NW
_K = 125
_NCHUNK = _EPW // _K
_NP = 10240
_RPT = _NP // _NS

_mesh = plsc.VectorSubcoreMesh(core_axis_name="c", subcore_axis_name="s")

_ZC = 80


@functools.partial(
    pl.kernel,
    out_type=jax.ShapeDtypeStruct((_NC, _NP, _D), jnp.float32),
    mesh=_mesh,
    scratch_types=[
        pltpu.VMEM((_NCHUNK, _K), jnp.int32),
        pltpu.VMEM((_K, _D), jnp.float32),
        pltpu.VMEM_SHARED((_NP, _D), jnp.float32),
        pltpu.SemaphoreType.DMA,
    ],
)
def _sc_degree(colw_hbm, dummy_hbm, out_hbm, idx_v, ones_v, acc_sh, sem):
    c = lax.axis_index("c")
    s = lax.axis_index("s")
    w = c * _NS + s

    pltpu.async_copy(colw_hbm.at[w], idx_v, sem)

    def _fill(val):
        def _f(i, carry):
            for j in range(_D // 16):
                ones_v[i, pl.ds(j * 16, 16)] = jnp.full((16,), val, jnp.float32)
            return carry
        return _f

    lax.fori_loop(0, _ZC, _fill(0.0), 0)

    def _zero_acc(k, carry):
        pltpu.sync_copy(ones_v.at[pl.ds(0, _ZC)],
                        acc_sh.at[pl.ds(s * _RPT + k * _ZC, _ZC)])
        return carry

    lax.fori_loop(0, _RPT // _ZC, _zero_acc, 0)
    lax.fori_loop(0, _K, _fill(1.0), 0)
    pltpu.make_async_copy(colw_hbm.at[w], idx_v, sem).wait()
    plsc.subcore_barrier()

    _W = 8

    def _step(i, carry):
        pltpu.async_copy(ones_v, acc_sh.at[idx_v.at[i]], sem, add=True)

        @pl.when(i >= _W)
        def _():
            pltpu.make_async_copy(dummy_hbm, ones_v, sem).wait()

        return carry

    lax.fori_loop(0, _NCHUNK, _step, 0)

    def _drain(i, carry):
        pltpu.make_async_copy(dummy_hbm, ones_v, sem).wait()
        return carry

    lax.fori_loop(0, _W, _drain, 0)
    plsc.subcore_barrier()
    pltpu.sync_copy(
        acc_sh.at[pl.ds(s * _RPT, _RPT)], out_hbm.at[c, pl.ds(s * _RPT, _RPT)]
    )


_SCH = 16
_NSEG = _NCHUNK // _SCH


@functools.partial(
    pl.kernel,
    out_type=jax.ShapeDtypeStruct((_NC, _NP, _D), jnp.float32),
    mesh=_mesh,
    scratch_types=[
        pltpu.VMEM((2, _SCH, _K), jnp.int32),
        pltpu.VMEM((2, _SCH, _K), jnp.int32),
        pltpu.VMEM((_K, _D), jnp.float32),
        pltpu.VMEM((_K, _D), jnp.float32),
        pltpu.VMEM_SHARED((_NP, _D), jnp.float32),
        pltpu.SemaphoreType.DMA,
        pltpu.SemaphoreType.DMA,
        pltpu.SemaphoreType.DMA,
        pltpu.SemaphoreType.DMA,
        pltpu.SemaphoreType.DMA,
    ],
)
def _sc_aggregate(hp_hbm, roww_hbm, colw_hbm, dummy_hbm, out_hbm, ri_v, ci_v,
                  buf0, buf1, acc_sh, sg0, sg1, ss0, ss1, si):
    c = lax.axis_index("c")
    s = lax.axis_index("s")
    w = c * _NS + s

    pltpu.async_copy(roww_hbm.at[w, pl.ds(0, _SCH)], ri_v.at[0], si)
    pltpu.async_copy(colw_hbm.at[w, pl.ds(0, _SCH)], ci_v.at[0], si)

    def _fz(i, carry):
        for j in range(_D // 16):
            buf0[i, pl.ds(j * 16, 16)] = jnp.zeros((16,), jnp.float32)
        return carry

    lax.fori_loop(0, _ZC, _fz, 0)

    def _zero_acc(k, carry):
        pltpu.sync_copy(buf0.at[pl.ds(0, _ZC)],
                        acc_sh.at[pl.ds(s * _RPT + k * _ZC, _ZC)])
        return carry

    lax.fori_loop(0, _RPT // _ZC, _zero_acc, 0)
    pltpu.make_async_copy(roww_hbm.at[w, pl.ds(0, _SCH)], ri_v.at[0], si).wait()
    pltpu.make_async_copy(colw_hbm.at[w, pl.ds(0, _SCH)], ci_v.at[0], si).wait()
    plsc.subcore_barrier()

    def _wait_gather(buf, sg):
        pltpu.make_async_copy(dummy_hbm, buf, sg).wait()

    def _drain_scatter(buf, ss):
        pltpu.make_async_copy(dummy_hbm, buf, ss).wait()

    def _seg(h, carry):
        par = lax.rem(h, 2)
        pnext = 1 - par

        @pl.when(h + 1 < _NSEG)
        def _():
            pltpu.async_copy(
                roww_hbm.at[w, pl.ds((h + 1) * _SCH, _SCH)], ri_v.at[pnext], si)
            pltpu.async_copy(
                colw_hbm.at[w, pl.ds((h + 1) * _SCH, _SCH)], ci_v.at[pnext], si)

        for j0 in range(0, _SCH, 2):
            j1 = j0 + 1
            if j0 >= 2:
                _drain_scatter(buf0, ss0)
            else:
                @pl.when(h > 0)
                def _():
                    _drain_scatter(buf0, ss0)
            pltpu.async_copy(hp_hbm.at[ri_v.at[par, j0]], buf0, sg0)
            if j0 >= 2:
                _drain_scatter(buf1, ss1)
            else:
                @pl.when(h > 0)
                def _():
                    _drain_scatter(buf1, ss1)
            pltpu.async_copy(hp_hbm.at[ri_v.at[par, j1]], buf1, sg1)
            _wait_gather(buf0, sg0)
            pltpu.async_copy(buf0, acc_sh.at[ci_v.at[par, j0]], ss0, add=True)
            _wait_gather(buf1, sg1)
            pltpu.async_copy(buf1, acc_sh.at[ci_v.at[par, j1]], ss1, add=True)

        @pl.when(h + 1 < _NSEG)
        def _():
            pltpu.make_async_copy(
                roww_hbm.at[w, pl.ds(0, _SCH)], ri_v.at[pnext], si).wait()
            pltpu.make_async_copy(
                colw_hbm.at[w, pl.ds(0, _SCH)], ci_v.at[pnext], si).wait()

        return carry

    lax.fori_loop(0, _NSEG, _seg, 0)
    _drain_scatter(buf0, ss0)
    _drain_scatter(buf1, ss1)
    plsc.subcore_barrier()
    pltpu.sync_copy(
        acc_sh.at[pl.ds(s * _RPT, _RPT)], out_hbm.at[c, pl.ds(s * _RPT, _RPT)]
    )


def _pre_body(p_ref, x_ref, we_ref, be_ref, w0_ref, dv_ref, hp_ref):
    d = p_ref[0, :_N] + p_ref[1, :_N]
    dv = jnp.where(d > 0, lax.rsqrt(d), 0.0)
    dv_ref[...] = dv
    h = lax.dot_general(x_ref[...], we_ref[...], (((1,), (1,)), ((), ())),
                        preferred_element_type=jnp.float32)
    h = jnp.maximum(h + be_ref[...], 0.0)
    hw = lax.dot_general(h, w0_ref[...], (((1,), (1,)), ((), ())),
                         preferred_element_type=jnp.float32)
    hp_ref[...] = hw * dv


def _pre_call(parts, x, we, be, w0):
    return pl.pallas_call(
        _pre_body,
        out_shape=[
            jax.ShapeDtypeStruct((_N, _D), jnp.float32),
            jax.ShapeDtypeStruct((_N, _D), jnp.float32),
        ],
    )(parts, x, we, be, w0)


def _layer_mm_body(p_ref, dv_ref, b_ref, r_ref, g_ref, be_ref, w_ref,
                   y_ref, hp_ref):
    y = (p_ref[0, :_N] + p_ref[1, :_N]) * dv_ref[...] + b_ref[...] + r_ref[...]
    y_ref[...] = y
    mu = jnp.mean(y, axis=0, keepdims=True)
    var = jnp.mean(y * y, axis=0, keepdims=True) - mu * mu
    h = jnp.maximum(
        (y - mu) * lax.rsqrt(var + _EPS) * g_ref[...] + be_ref[...], 0.0)
    hw = lax.dot_general(h, w_ref[...], (((1,), (1,)), ((), ())),
                         preferred_element_type=jnp.float32)
    hp_ref[...] = hw * dv_ref[...]


def _layer_mm0_body(p_ref, dv_ref, b_ref, g_ref, be_ref, w_ref, y_ref, hp_ref):
    y = (p_ref[0, :_N] + p_ref[1, :_N]) * dv_ref[...] + b_ref[...]
    y_ref[...] = y
    mu = jnp.mean(y, axis=0, keepdims=True)
    var = jnp.mean(y * y, axis=0, keepdims=True) - mu * mu
    h = jnp.maximum(
        (y - mu) * lax.rsqrt(var + _EPS) * g_ref[...] + be_ref[...], 0.0)
    hw = lax.dot_general(h, w_ref[...], (((1,), (1,)), ((), ())),
                         preferred_element_type=jnp.float32)
    hp_ref[...] = hw * dv_ref[...]


def _layer_call(parts, dinv, b, res, g, be, w_next):
    out_shape = [
        jax.ShapeDtypeStruct((_N, _D), jnp.float32),
        jax.ShapeDtypeStruct((_N, _D), jnp.float32),
    ]
    if res is None:
        return pl.pallas_call(_layer_mm0_body, out_shape=out_shape)(
            parts, dinv, b, g, be, w_next)
    return pl.pallas_call(_layer_mm_body, out_shape=out_shape)(
        parts, dinv, b, res, g, be, w_next)


def _layer_pred_body(p_ref, dv_ref, b_ref, r_ref, g_ref, be_ref, w_ref,
                     bp_ref, o_ref):
    y = (p_ref[0, :_N] + p_ref[1, :_N]) * dv_ref[...] + b_ref[...] + r_ref[...]
    mu = jnp.mean(y, axis=0, keepdims=True)
    var = jnp.mean(y * y, axis=0, keepdims=True) - mu * mu
    h = jnp.maximum(
        (y - mu) * lax.rsqrt(var + _EPS) * g_ref[...] + be_ref[...], 0.0)
    hw = lax.dot_general(h, w_ref[...], (((1,), (1,)), ((), ())),
                         preferred_element_type=jnp.float32)
    o_ref[...] = hw + bp_ref[...]


def _layer_pred_call(parts, dinv, b, res, g, be, wp, bp):
    return pl.pallas_call(
        _layer_pred_body,
        out_shape=jax.ShapeDtypeStruct((_N, _D), jnp.float32),
    )(parts, dinv, b, res, g, be, wp, bp)


def kernel(x, edge_index, W_enc, b_enc, Ws, bs, gammas, betas, W_pred, b_pred):
    roww = edge_index[0].reshape(_NW, _NCHUNK, _K)
    colw = edge_index[1].reshape(_NW, _NCHUNK, _K)

    dummy = jnp.zeros((_K, _D), jnp.float32)
    deg_parts = _sc_degree(colw, dummy)
    dinv, hp = _pre_call(deg_parts, x, W_enc, b_enc.reshape(1, _D), Ws[0])

    wp_pad = jnp.zeros((_D, _D), jnp.float32).at[:_C].set(W_pred)
    bp_pad = jnp.zeros((1, _D), jnp.float32).at[0, :_C].set(b_pred)

    y_last = None
    out = None
    for i in range(_L):
        parts = _sc_aggregate(hp, roww, colw, dummy)
        b = bs[i].reshape(1, _D)
        g = gammas[i].reshape(1, _D)
        be = betas[i].reshape(1, _D)
        if i < _L - 1:
            y, hp = _layer_call(parts, dinv, b, y_last, g, be, Ws[i + 1])
            y_last = y
        else:
            out = _layer_pred_call(parts, dinv, b, y_last, g, be, wp_pad,
                                   bp_pad)
    return out[:, :_C]
